# Initial kernel scaffold; baseline (speedup 1.0000x reference)
import jax
import jax.numpy as jnp
from jax.experimental import pallas as pl


def kernel(pointcloud, keys, values):
    raise NotImplementedError("write your pallas kernel here")


# speedup vs baseline: 1.1923x; 1.1923x over previous
import functools

import jax
import jax.numpy as jnp
from jax import lax
from jax.experimental import pallas as pl
from jax.experimental.pallas import tpu as pltpu
from jax.experimental.pallas import tpu_sc as plsc

_B, _N, _K, _D = 4, 4096, 1024, 64
_P = _B * _N
_ROW_BLK = 2048

_NC, _NS = 2, 16
_NW = _NC * _NS
_IDXW = 128
_CHUNKS = _P // (_NW * _IDXW)


def _argmin_body(px_ref, py_ref, pz_ref, kx_ref, ky_ref, kz_ref, ids_ref):
    px = px_ref[...]
    py = py_ref[...]
    pz = pz_ref[...]
    kx = kx_ref[...]
    ky = ky_ref[...]
    kz = kz_ref[...]
    d0 = px - kx
    acc = d0 * d0
    d1 = py - ky
    acc = acc + d1 * d1
    d2 = pz - kz
    acc = acc + d2 * d2
    m = jnp.min(acc, axis=1, keepdims=True)
    io = lax.broadcasted_iota(jnp.int32, (_ROW_BLK, _K), 1)
    idx = jnp.min(jnp.where(acc <= m, io, _K), axis=1, keepdims=True)
    ids_ref[...] = idx


def _tc_argmin(px, py, pz, kx, ky, kz):
    return pl.pallas_call(
        _argmin_body,
        grid=(_P // _ROW_BLK,),
        in_specs=[
            pl.BlockSpec((_ROW_BLK, 1), lambda i: (i, 0)),
            pl.BlockSpec((_ROW_BLK, 1), lambda i: (i, 0)),
            pl.BlockSpec((_ROW_BLK, 1), lambda i: (i, 0)),
            pl.BlockSpec((1, _K), lambda i: (0, 0)),
            pl.BlockSpec((1, _K), lambda i: (0, 0)),
            pl.BlockSpec((1, _K), lambda i: (0, 0)),
        ],
        out_specs=pl.BlockSpec((_ROW_BLK, 1), lambda i: (i, 0)),
        out_shape=jax.ShapeDtypeStruct((_P, 1), jnp.int32),
    )(px, py, pz, kx, ky, kz)


@functools.partial(
    pl.kernel,
    out_type=jax.ShapeDtypeStruct((_P // _IDXW, _IDXW, _D), jnp.float32),
    mesh=plsc.VectorSubcoreMesh(core_axis_name="c", subcore_axis_name="s"),
    scratch_types=[
        pltpu.VMEM((_CHUNKS, _IDXW), jnp.int32),
        pltpu.VMEM((_CHUNKS, _IDXW, _D), jnp.float32),
        pltpu.SemaphoreType.DMA,
    ],
    compiler_params=pltpu.CompilerParams(use_tc_tiling_on_sc=False),
)
def _sc_gather(values_hbm, idx_hbm, out_hbm, idx_v, rows_v, sem):
    wid = lax.axis_index("s") * _NC + lax.axis_index("c")
    base = wid * _CHUNKS
    pltpu.sync_copy(idx_hbm.at[pl.ds(base, _CHUNKS)], idx_v)
    copies = [
        pltpu.async_copy(values_hbm.at[idx_v.at[j]], rows_v.at[j], sem)
        for j in range(_CHUNKS)
    ]
    for c in copies:
        c.wait()
    pltpu.sync_copy(rows_v, out_hbm.at[pl.ds(base, _CHUNKS)])


def kernel(pointcloud, keys, values):
    pts = pointcloud.reshape(_P, 3)
    px, py, pz = pts[:, 0:1], pts[:, 1:2], pts[:, 2:3]
    kx, ky, kz = keys[:, 0:1].T, keys[:, 1:2].T, keys[:, 2:3].T
    ids = _tc_argmin(px, py, pz, kx, ky, kz)
    idx2d = ids.reshape(_P // _IDXW, _IDXW)
    feats = _sc_gather(values, idx2d)
    return feats.reshape(_B, _N, _D), pointcloud


# speedup vs baseline: 1.4929x; 1.2522x over previous
import functools

import jax
import jax.numpy as jnp
from jax import lax
from jax.experimental import pallas as pl
from jax.experimental.pallas import tpu as pltpu
from jax.experimental.pallas import tpu_sc as plsc

_B, _N, _K, _D = 4, 4096, 1024, 64
_P = _B * _N
_ROW_BLK = 2048
_STEPS = _P // _ROW_BLK

_NC, _NS = 2, 16
_NW = _NC * _NS
_IDXW = 128
_CHUNKS = _P // (_NW * _IDXW)
_ROWS_W = _CHUNKS * _IDXW
_W_PER_B = _N // _ROWS_W


def _argmin_body(pts_ref, keys_ref, ids_ref):
    px = pts_ref[0:1, :]
    py = pts_ref[1:2, :]
    pz = pts_ref[2:3, :]
    kx = keys_ref[:, 0:1]
    ky = keys_ref[:, 1:2]
    kz = keys_ref[:, 2:3]
    d0 = px - kx
    acc = d0 * d0
    d1 = py - ky
    acc = acc + d1 * d1
    d2 = pz - kz
    acc = acc + d2 * d2
    m = jnp.min(acc, axis=0, keepdims=True)
    io = lax.broadcasted_iota(jnp.int32, (_K, _ROW_BLK), 0)
    idx = jnp.min(jnp.where(acc <= m, io, _K), axis=0, keepdims=True)
    ids_ref[...] = idx.reshape(1, 1, _ROW_BLK)


def _tc_argmin(pts_t, keys):
    return pl.pallas_call(
        _argmin_body,
        grid=(_STEPS,),
        in_specs=[
            pl.BlockSpec((3, _ROW_BLK), lambda i: (0, i)),
            pl.BlockSpec((_K, 3), lambda i: (0, 0)),
        ],
        out_specs=pl.BlockSpec((1, 1, _ROW_BLK), lambda i: (i, 0, 0)),
        out_shape=jax.ShapeDtypeStruct((_STEPS, 1, _ROW_BLK), jnp.int32),
    )(pts_t, keys)


@functools.partial(
    pl.kernel,
    out_type=jax.ShapeDtypeStruct((_B, _N, _D), jnp.float32),
    mesh=plsc.VectorSubcoreMesh(core_axis_name="c", subcore_axis_name="s"),
    scratch_types=[
        pltpu.VMEM((_CHUNKS, _IDXW), jnp.int32),
        pltpu.VMEM((_ROWS_W, _D), jnp.float32),
        pltpu.SemaphoreType.DMA,
    ],
    compiler_params=pltpu.CompilerParams(use_tc_tiling_on_sc=False),
)
def _sc_gather(values_hbm, idx_hbm, out_hbm, idx_v, rows_v, sem):
    wid = lax.axis_index("s") * _NC + lax.axis_index("c")
    pltpu.sync_copy(idx_hbm.at[pl.ds(wid * _CHUNKS, _CHUNKS)], idx_v)
    copies = [
        pltpu.async_copy(
            values_hbm.at[idx_v.at[j]],
            rows_v.at[pl.ds(j * _IDXW, _IDXW)],
            sem,
        )
        for j in range(_CHUNKS)
    ]
    for c in copies:
        c.wait()
    b = wid // _W_PER_B
    off = (wid % _W_PER_B) * _ROWS_W
    pltpu.sync_copy(rows_v, out_hbm.at[b, pl.ds(off, _ROWS_W)])


def kernel(pointcloud, keys, values):
    pts_t = pointcloud.reshape(_P, 3).T
    ids = _tc_argmin(pts_t, keys)
    idx2d = ids.reshape(_P // _IDXW, _IDXW)
    feats = _sc_gather(values, idx2d)
    return feats, pointcloud


# speedup vs baseline: 1.5863x; 1.0626x over previous
import functools

import jax
import jax.numpy as jnp
from jax import lax
from jax.experimental import pallas as pl
from jax.experimental.pallas import tpu as pltpu
from jax.experimental.pallas import tpu_sc as plsc

_B, _N, _K, _D = 4, 4096, 1024, 64
_P = _B * _N
_ROW_BLK = 2048
_STEPS = _P // _ROW_BLK

_NC, _NS = 2, 16
_NW = _NC * _NS
_IDXW = 128
_CHUNKS = _P // (_NW * _IDXW)
_ROWS_W = _CHUNKS * _IDXW
_W_PER_B = _N // _ROWS_W


def _argmin_body(pts_ref, keys_ref, ids_ref):
    px = pts_ref[0:1, :]
    py = pts_ref[1:2, :]
    pz = pts_ref[2:3, :]
    kx = keys_ref[:, 0:1]
    ky = keys_ref[:, 1:2]
    kz = keys_ref[:, 2:3]
    d0 = px - kx
    acc = d0 * d0
    d1 = py - ky
    acc = acc + d1 * d1
    d2 = pz - kz
    acc = acc + d2 * d2
    m = jnp.min(acc, axis=0, keepdims=True)
    io = lax.broadcasted_iota(jnp.int32, (_K, _ROW_BLK), 0)
    idx = jnp.min(jnp.where(acc <= m, io, _K), axis=0, keepdims=True)
    ids_ref[...] = idx.reshape(1, 1, _ROW_BLK)


def _tc_argmin(pts_t, keys):
    return pl.pallas_call(
        _argmin_body,
        grid=(_STEPS,),
        in_specs=[
            pl.BlockSpec((3, _ROW_BLK), lambda i: (0, i)),
            pl.BlockSpec((_K, 3), lambda i: (0, 0)),
        ],
        out_specs=pl.BlockSpec((1, 1, _ROW_BLK), lambda i: (i, 0, 0)),
        out_shape=jax.ShapeDtypeStruct((_STEPS, 1, _ROW_BLK), jnp.int32),
    )(pts_t, keys)


_DBLK = _D // (_NW // _B)
_L = 16


@functools.partial(
    pl.kernel,
    out_type=jax.ShapeDtypeStruct((_B, _D, _N), jnp.float32),
    mesh=plsc.VectorSubcoreMesh(core_axis_name="c", subcore_axis_name="s"),
    scratch_types=[
        pltpu.VMEM((_DBLK, _K), jnp.float32),
        pltpu.VMEM((_N,), jnp.int32),
        pltpu.VMEM((_DBLK, _N), jnp.float32),
    ],
    compiler_params=pltpu.CompilerParams(
        use_tc_tiling_on_sc=False, needs_layout_passes=False
    ),
)
def _sc_gather_t(values_t_hbm, idx_hbm, out_hbm, vt_v, ids_v, out_v):
    wid = lax.axis_index("s") * _NC + lax.axis_index("c")
    b = wid // (_D // _DBLK)
    t = wid % (_D // _DBLK)
    pltpu.sync_copy(values_t_hbm.at[pl.ds(t * _DBLK, _DBLK)], vt_v)
    pltpu.sync_copy(idx_hbm.at[pl.ds(b * _N, _N)], ids_v)

    def body(i, _):
        n0 = i * _L
        id16 = ids_v[pl.ds(n0, _L)]
        for d in range(_DBLK):
            vals = plsc.load_gather(
                vt_v, [jnp.full((_L,), d, jnp.int32), id16]
            )
            out_v[d, pl.ds(n0, _L)] = vals
        return _

    lax.fori_loop(0, _N // _L, body, None)
    pltpu.sync_copy(out_v, out_hbm.at[b, pl.ds(t * _DBLK, _DBLK)])


def kernel(pointcloud, keys, values):
    pts_t = jnp.transpose(pointcloud, (2, 0, 1)).reshape(3, _P)
    ids = _tc_argmin(pts_t, keys)
    feats_t = _sc_gather_t(values.T, ids.reshape(_P))
    return jnp.transpose(feats_t, (0, 2, 1)), pointcloud
